# Initial kernel scaffold; baseline (speedup 1.0000x reference)
import jax
import jax.numpy as jnp
from jax.experimental import pallas as pl


def kernel(x, bias, gamma, beta):
    raise NotImplementedError("write your pallas kernel here")


# speedup vs baseline: 2.8495x; 2.8495x over previous
import jax
import jax.numpy as jnp
from jax.experimental import pallas as pl
from jax.experimental.pallas import tpu as pltpu

_NUM_GROUPS = 64
_EPS = 1e-5

_MISH_COEFS = (
    0.6080945134162903,
    0.3328809440135956,
    -0.010158239863812923,
    -0.048012491315603256,
    -0.0033194711431860924,
    0.007374798413366079,
    0.0014715775614604354,
    -0.0009952515829354525,
    -0.0003400757268536836,
    0.00011020980309695005,
    4.37905327999033e-05,
    -7.80593018134823e-06,
)


def _clamped_mish_poly(c):
    acc = jnp.full_like(c, _MISH_COEFS[-1])
    for coef in _MISH_COEFS[-2::-1]:
        acc = acc * c + coef
    return acc * c


def _split_hi_lo(v):
    hi = v.astype(jnp.bfloat16)
    lo = (v - hi.astype(jnp.float32)).astype(jnp.bfloat16)
    return hi, lo


def _fused_body(x_ref, b_ref, g_ref, bt_ref, m_ref, mt_ref, o_ref):
    gs = x_ref.shape[1] // _NUM_GROUPS

    t = jnp.clip(x_ref[...] + b_ref[...], -1.0, 1.0)
    v = _clamped_mish_poly(t)
    q = v * v

    m = m_ref[...]
    v_hi, v_lo = _split_hi_lo(v)
    q_hi, q_lo = _split_hi_lo(q)
    s1 = (jnp.dot(v_hi, m, preferred_element_type=jnp.float32)
          + jnp.dot(v_lo, m, preferred_element_type=jnp.float32))
    s2 = (jnp.dot(q_hi, m, preferred_element_type=jnp.float32)
          + jnp.dot(q_lo, m, preferred_element_type=jnp.float32))

    inv_gs = 1.0 / gs
    mean = s1 * inv_gs
    var = s2 * inv_gs - mean * mean
    inv_std = jax.lax.rsqrt(var + _EPS)

    mt = mt_ref[...]
    m_hi, m_lo = _split_hi_lo(mean)
    i_hi, i_lo = _split_hi_lo(inv_std)
    mean_b = jnp.dot(jnp.concatenate([m_hi, m_lo], axis=1), mt,
                     preferred_element_type=jnp.float32)
    inv_b = jnp.dot(jnp.concatenate([i_hi, i_lo], axis=1), mt,
                    preferred_element_type=jnp.float32)

    o_ref[...] = (v - mean_b) * inv_b * g_ref[...] + bt_ref[...]


def kernel(x, bias, gamma, beta):
    n, c = x.shape
    g = _NUM_GROUPS
    block_n = 256

    chan = jnp.arange(c, dtype=jnp.int32) // (c // g)
    m = (chan[:, None] == jnp.arange(g, dtype=jnp.int32)[None, :]).astype(
        jnp.bfloat16)
    mt = jnp.concatenate([m.T, m.T], axis=0)

    grid = (n // block_n,)
    row_spec = pl.BlockSpec((block_n, c), lambda i: (i, 0))
    param_spec = lambda shape: pl.BlockSpec(shape, lambda i: (0, 0))

    return pl.pallas_call(
        _fused_body,
        grid=grid,
        in_specs=[
            row_spec,
            param_spec((1, c)),
            param_spec((1, c)),
            param_spec((1, c)),
            param_spec((c, g)),
            param_spec((2 * g, c)),
        ],
        out_specs=row_spec,
        out_shape=jax.ShapeDtypeStruct((n, c), jnp.float32),
        compiler_params=pltpu.CompilerParams(
            dimension_semantics=("parallel",),
        ),
    )(x, bias.reshape(1, c), gamma.reshape(1, c), beta.reshape(1, c), m, mt)


# speedup vs baseline: 3.3689x; 1.1823x over previous
import jax
import jax.numpy as jnp
from jax.experimental import pallas as pl
from jax.experimental.pallas import tpu as pltpu

_NUM_GROUPS = 64
_EPS = 1e-5

_MISH_COEFS = (
    0.6081030368804932,
    0.3328776955604553,
    -0.010273046791553497,
    -0.0479687862098217,
    -0.0028912960551679134,
    0.007214798592031002,
    0.0008541956776753068,
    -0.0007736249826848507,
)


def _clamped_mish_poly(c):
    acc = jnp.full_like(c, _MISH_COEFS[-1])
    for coef in _MISH_COEFS[-2::-1]:
        acc = acc * c + coef
    return acc * c


def _split_hi_lo(v):
    hi = v.astype(jnp.bfloat16)
    lo = (v - hi.astype(jnp.float32)).astype(jnp.bfloat16)
    return hi, lo


def _fused_body(x_ref, b_ref, g_ref, bt_ref, m_ref, mt_ref, o_ref):
    gs = x_ref.shape[1] // _NUM_GROUPS

    t = jnp.clip(x_ref[...] + b_ref[...], -1.0, 1.0)
    v = _clamped_mish_poly(t)
    q = v * v

    m = m_ref[...]
    v_hi, v_lo = _split_hi_lo(v)
    q_hi, q_lo = _split_hi_lo(q)
    s1 = (jnp.dot(v_hi, m, preferred_element_type=jnp.float32)
          + jnp.dot(v_lo, m, preferred_element_type=jnp.float32))
    s2 = (jnp.dot(q_hi, m, preferred_element_type=jnp.float32)
          + jnp.dot(q_lo, m, preferred_element_type=jnp.float32))

    inv_gs = 1.0 / gs
    mean = s1 * inv_gs
    var = s2 * inv_gs - mean * mean
    inv_std = jax.lax.rsqrt(var + _EPS)

    mt = mt_ref[...]
    m_hi, m_lo = _split_hi_lo(mean)
    i_hi, i_lo = _split_hi_lo(inv_std)
    mean_b = jnp.dot(jnp.concatenate([m_hi, m_lo], axis=1), mt,
                     preferred_element_type=jnp.float32)
    inv_b = jnp.dot(jnp.concatenate([i_hi, i_lo], axis=1), mt,
                    preferred_element_type=jnp.float32)

    o_ref[...] = (v - mean_b) * inv_b * g_ref[...] + bt_ref[...]


def kernel(x, bias, gamma, beta):
    n, c = x.shape
    g = _NUM_GROUPS
    block_n = 256

    chan = jnp.arange(c, dtype=jnp.int32) // (c // g)
    m = (chan[:, None] == jnp.arange(g, dtype=jnp.int32)[None, :]).astype(
        jnp.bfloat16)
    mt = jnp.concatenate([m.T, m.T], axis=0)

    grid = (n // block_n,)
    row_spec = pl.BlockSpec((block_n, c), lambda i: (i, 0))
    param_spec = lambda shape: pl.BlockSpec(shape, lambda i: (0, 0))

    return pl.pallas_call(
        _fused_body,
        grid=grid,
        in_specs=[
            row_spec,
            param_spec((1, c)),
            param_spec((1, c)),
            param_spec((1, c)),
            param_spec((c, g)),
            param_spec((2 * g, c)),
        ],
        out_specs=row_spec,
        out_shape=jax.ShapeDtypeStruct((n, c), jnp.float32),
        compiler_params=pltpu.CompilerParams(
            dimension_semantics=("parallel",),
        ),
    )(x, bias.reshape(1, c), gamma.reshape(1, c), beta.reshape(1, c), m, mt)
